# Initial kernel scaffold; baseline (speedup 1.0000x reference)
import jax
import jax.numpy as jnp
from jax.experimental import pallas as pl


def kernel(H_v, batch):
    raise NotImplementedError("write your pallas kernel here")


# speedup vs baseline: 4.0956x; 4.0956x over previous
import jax
import jax.numpy as jnp
from jax import lax
from jax.experimental import pallas as pl
from jax.experimental.pallas import tpu as pltpu
from jax.experimental.pallas import tpu_sc as plsc

N = 320000
D = 128
S = 512
NC = 2
NS = 16
NW = NC * NS
RPW = N // NW
K = 100
G = RPW // K
CHUNK = 400
GPC = CHUNK // K
NCHUNK = RPW // CHUNK
CW = 128
ROWS_PER_TILE_INIT = S // NS


def _sc_segment_sum(H_v, bidx, zacc, zcnt, ones):
    mesh = plsc.VectorSubcoreMesh(core_axis_name="c", subcore_axis_name="s")

    def body(hv, idx_hbm, zacc_hbm, zcnt_hbm, ones_hbm,
             psum, pcnt, idx_v, buf_v, ones_v, acc_sh, cnt_sh):
        cid = lax.axis_index("c")
        sid = lax.axis_index("s")
        wid = cid * NS + sid
        row_base = wid * RPW

        pltpu.sync_copy(idx_hbm.at[wid], idx_v)
        pltpu.sync_copy(ones_hbm, ones_v)
        r0 = sid * ROWS_PER_TILE_INIT
        pltpu.sync_copy(zacc_hbm.at[pl.ds(r0, ROWS_PER_TILE_INIT)],
                        acc_sh.at[pl.ds(r0, ROWS_PER_TILE_INIT)])
        pltpu.sync_copy(zcnt_hbm.at[pl.ds(r0, ROWS_PER_TILE_INIT)],
                        cnt_sh.at[pl.ds(r0, ROWS_PER_TILE_INIT)])
        plsc.subcore_barrier()

        def chunk_body(j, carry):
            pltpu.sync_copy(hv.at[pl.ds(row_base + j * CHUNK, CHUNK)], buf_v)
            for g in range(GPC):
                jg = j * GPC + g
                pltpu.sync_copy(buf_v.at[pl.ds(g * K, K)],
                                acc_sh.at[idx_v.at[jg]], add=True)
                pltpu.sync_copy(ones_v, cnt_sh.at[idx_v.at[jg]], add=True)
            return carry

        lax.fori_loop(0, NCHUNK, chunk_body, 0)
        plsc.subcore_barrier()

        pltpu.sync_copy(acc_sh.at[pl.ds(r0, ROWS_PER_TILE_INIT)],
                        psum.at[cid, pl.ds(r0, ROWS_PER_TILE_INIT)])
        pltpu.sync_copy(cnt_sh.at[pl.ds(r0, ROWS_PER_TILE_INIT)],
                        pcnt.at[cid, pl.ds(r0, ROWS_PER_TILE_INIT)])

    fn = pl.kernel(
        body,
        out_type=(
            jax.ShapeDtypeStruct((NC, S, D), jnp.float32),
            jax.ShapeDtypeStruct((NC, S, CW), jnp.float32),
        ),
        mesh=mesh,
        scratch_types=(
            pltpu.VMEM((G, K), jnp.int32),
            pltpu.VMEM((CHUNK, D), jnp.float32),
            pltpu.VMEM((K, CW), jnp.float32),
            pltpu.VMEM_SHARED((S, D), jnp.float32),
            pltpu.VMEM_SHARED((S, CW), jnp.float32),
        ),
    )
    return fn(H_v, bidx, zacc, zcnt, ones)


def _finalize_body(ps_ref, pc_ref, out_ref):
    sums = ps_ref[0] + ps_ref[1]
    counts = jnp.maximum((pc_ref[0] + pc_ref[1])[:, 0:1], 1.0)
    out_ref[...] = sums / counts


def kernel(H_v, batch):
    bidx = batch.reshape(NW, G, K)
    zacc = jnp.zeros((S, D), jnp.float32)
    zcnt = jnp.zeros((S, CW), jnp.float32)
    ones = jnp.ones((K, CW), jnp.float32)
    psum, pcnt = _sc_segment_sum(H_v, bidx, zacc, zcnt, ones)
    return pl.pallas_call(
        _finalize_body,
        out_shape=jax.ShapeDtypeStruct((S, D), jnp.float32),
    )(psum, pcnt)


# speedup vs baseline: 4.3715x; 1.0674x over previous
import jax
import jax.numpy as jnp
from jax import lax
from jax.experimental import pallas as pl
from jax.experimental.pallas import tpu as pltpu
from jax.experimental.pallas import tpu_sc as plsc

N = 320000
D = 128
S = 512
NC = 2
NS = 16
NW = NC * NS
RPW = N // NW
K = 100
G = RPW // K
CHUNK = 200
GPC = CHUNK // K
NCHUNK = RPW // CHUNK
CW = 128
ROWS_PER_TILE_INIT = S // NS


def _sc_segment_sum(H_v, bidx, zacc, zcnt, ones):
    mesh = plsc.VectorSubcoreMesh(core_axis_name="c", subcore_axis_name="s")

    def body(hv, idx_hbm, zacc_hbm, zcnt_hbm, ones_hbm,
             psum, pcnt, idx_v, buf0, buf1, ones_v, acc_sh, cnt_sh,
             sem0, sem1):
        cid = lax.axis_index("c")
        sid = lax.axis_index("s")
        wid = cid * NS + sid
        row_base = wid * RPW

        pltpu.sync_copy(idx_hbm.at[wid], idx_v)
        pltpu.sync_copy(ones_hbm, ones_v)
        r0 = sid * ROWS_PER_TILE_INIT
        pltpu.sync_copy(zacc_hbm.at[pl.ds(r0, ROWS_PER_TILE_INIT)],
                        acc_sh.at[pl.ds(r0, ROWS_PER_TILE_INIT)])
        pltpu.sync_copy(zcnt_hbm.at[pl.ds(r0, ROWS_PER_TILE_INIT)],
                        cnt_sh.at[pl.ds(r0, ROWS_PER_TILE_INIT)])
        plsc.subcore_barrier()

        def hv_chunk(c):
            return hv.at[pl.ds(row_base + c * CHUNK, CHUNK)]

        def issue(c, buf, sem):
            pltpu.async_copy(hv_chunk(c), buf, sem)

        def wait(c, buf, sem):
            pltpu.make_async_copy(hv_chunk(c), buf, sem).wait()

        def scatter(c, buf):
            for g in range(GPC):
                jg = c * GPC + g
                pltpu.sync_copy(buf.at[pl.ds(g * K, K)],
                                acc_sh.at[idx_v.at[jg]], add=True)
                pltpu.sync_copy(ones_v, cnt_sh.at[idx_v.at[jg]], add=True)

        issue(0, buf0, sem0)

        def pair_body(j, carry):
            c0 = 2 * j
            wait(c0, buf0, sem0)
            issue(c0 + 1, buf1, sem1)
            scatter(c0, buf0)
            wait(c0 + 1, buf1, sem1)
            issue(c0 + 2, buf0, sem0)
            scatter(c0 + 1, buf1)
            return carry

        lax.fori_loop(0, (NCHUNK - 1) // 2, pair_body, 0)
        if NCHUNK % 2:
            wait(NCHUNK - 1, buf0, sem0)
            scatter(NCHUNK - 1, buf0)
        else:
            wait(NCHUNK - 2, buf0, sem0)
            issue(NCHUNK - 1, buf1, sem1)
            scatter(NCHUNK - 2, buf0)
            wait(NCHUNK - 1, buf1, sem1)
            scatter(NCHUNK - 1, buf1)
        plsc.subcore_barrier()

        pltpu.sync_copy(acc_sh.at[pl.ds(r0, ROWS_PER_TILE_INIT)],
                        psum.at[cid, pl.ds(r0, ROWS_PER_TILE_INIT)])
        pltpu.sync_copy(cnt_sh.at[pl.ds(r0, ROWS_PER_TILE_INIT)],
                        pcnt.at[cid, pl.ds(r0, ROWS_PER_TILE_INIT)])

    fn = pl.kernel(
        body,
        out_type=(
            jax.ShapeDtypeStruct((NC, S, D), jnp.float32),
            jax.ShapeDtypeStruct((NC, S, CW), jnp.float32),
        ),
        mesh=mesh,
        scratch_types=(
            pltpu.VMEM((G, K), jnp.int32),
            pltpu.VMEM((CHUNK, D), jnp.float32),
            pltpu.VMEM((CHUNK, D), jnp.float32),
            pltpu.VMEM((K, CW), jnp.float32),
            pltpu.VMEM_SHARED((S, D), jnp.float32),
            pltpu.VMEM_SHARED((S, CW), jnp.float32),
            pltpu.SemaphoreType.DMA,
            pltpu.SemaphoreType.DMA,
        ),
    )
    return fn(H_v, bidx, zacc, zcnt, ones)


def _finalize_body(ps_ref, pc_ref, out_ref):
    sums = ps_ref[0] + ps_ref[1]
    counts = jnp.maximum((pc_ref[0] + pc_ref[1])[:, 0:1], 1.0)
    out_ref[...] = sums / counts


def kernel(H_v, batch):
    bidx = batch.reshape(NW, G, K)
    zacc = jnp.zeros((S, D), jnp.float32)
    zcnt = jnp.zeros((S, CW), jnp.float32)
    ones = jnp.ones((K, CW), jnp.float32)
    psum, pcnt = _sc_segment_sum(H_v, bidx, zacc, zcnt, ones)
    return pl.pallas_call(
        _finalize_body,
        out_shape=jax.ShapeDtypeStruct((S, D), jnp.float32),
    )(psum, pcnt)


# speedup vs baseline: 4.3722x; 1.0001x over previous
import jax
import jax.numpy as jnp
from jax import lax
from jax.experimental import pallas as pl
from jax.experimental.pallas import tpu as pltpu
from jax.experimental.pallas import tpu_sc as plsc

N = 320000
D = 128
S = 512
NC = 2
NS = 16
NW = NC * NS
RPW = N // NW
K = 100
G = RPW // K
CHUNK = 200
GPC = CHUNK // K
NCHUNK = RPW // CHUNK
CW = 128
ROWS_PER_TILE_INIT = S // NS


def _sc_segment_sum(H_v, bidx, zacc, zcnt, ones):
    mesh = plsc.VectorSubcoreMesh(core_axis_name="c", subcore_axis_name="s")

    def body(hv, idx_hbm, zacc_hbm, zcnt_hbm, ones_hbm,
             psum, pcnt, idx_v, buf0, buf1, ones_v, acc_sh, cnt_sh,
             sem0, sem1, semS):
        cid = lax.axis_index("c")
        sid = lax.axis_index("s")
        wid = cid * NS + sid
        row_base = wid * RPW

        pltpu.sync_copy(idx_hbm.at[wid], idx_v)
        pltpu.sync_copy(ones_hbm, ones_v)
        r0 = sid * ROWS_PER_TILE_INIT
        pltpu.sync_copy(zacc_hbm.at[pl.ds(r0, ROWS_PER_TILE_INIT)],
                        acc_sh.at[pl.ds(r0, ROWS_PER_TILE_INIT)])
        pltpu.sync_copy(zcnt_hbm.at[pl.ds(r0, ROWS_PER_TILE_INIT)],
                        cnt_sh.at[pl.ds(r0, ROWS_PER_TILE_INIT)])
        plsc.subcore_barrier()

        def hv_chunk(c):
            return hv.at[pl.ds(row_base + c * CHUNK, CHUNK)]

        def issue(c, buf, sem):
            pltpu.async_copy(hv_chunk(c), buf, sem)

        def wait(c, buf, sem):
            pltpu.make_async_copy(hv_chunk(c), buf, sem).wait()

        def scatter(c, buf):
            for g in range(GPC):
                jg = c * GPC + g
                pltpu.async_copy(buf.at[pl.ds(g * K, K)],
                                 acc_sh.at[idx_v.at[jg]], semS, add=True)
                pltpu.async_copy(ones_v, cnt_sh.at[idx_v.at[jg]], semS,
                                 add=True)
                pltpu.make_async_copy(
                    buf.at[pl.ds(g * K, K)], acc_sh.at[idx_v.at[jg]],
                    semS).wait()
                pltpu.make_async_copy(
                    ones_v, cnt_sh.at[idx_v.at[jg]], semS).wait()

        issue(0, buf0, sem0)

        def pair_body(j, carry):
            c0 = 2 * j
            wait(c0, buf0, sem0)
            issue(c0 + 1, buf1, sem1)
            scatter(c0, buf0)
            wait(c0 + 1, buf1, sem1)
            issue(c0 + 2, buf0, sem0)
            scatter(c0 + 1, buf1)
            return carry

        lax.fori_loop(0, (NCHUNK - 1) // 2, pair_body, 0)
        if NCHUNK % 2:
            wait(NCHUNK - 1, buf0, sem0)
            scatter(NCHUNK - 1, buf0)
        else:
            wait(NCHUNK - 2, buf0, sem0)
            issue(NCHUNK - 1, buf1, sem1)
            scatter(NCHUNK - 2, buf0)
            wait(NCHUNK - 1, buf1, sem1)
            scatter(NCHUNK - 1, buf1)
        plsc.subcore_barrier()

        pltpu.sync_copy(acc_sh.at[pl.ds(r0, ROWS_PER_TILE_INIT)],
                        psum.at[cid, pl.ds(r0, ROWS_PER_TILE_INIT)])
        pltpu.sync_copy(cnt_sh.at[pl.ds(r0, ROWS_PER_TILE_INIT)],
                        pcnt.at[cid, pl.ds(r0, ROWS_PER_TILE_INIT)])

    fn = pl.kernel(
        body,
        out_type=(
            jax.ShapeDtypeStruct((NC, S, D), jnp.float32),
            jax.ShapeDtypeStruct((NC, S, CW), jnp.float32),
        ),
        mesh=mesh,
        scratch_types=(
            pltpu.VMEM((G, K), jnp.int32),
            pltpu.VMEM((CHUNK, D), jnp.float32),
            pltpu.VMEM((CHUNK, D), jnp.float32),
            pltpu.VMEM((K, CW), jnp.float32),
            pltpu.VMEM_SHARED((S, D), jnp.float32),
            pltpu.VMEM_SHARED((S, CW), jnp.float32),
            pltpu.SemaphoreType.DMA,
            pltpu.SemaphoreType.DMA,
            pltpu.SemaphoreType.DMA,
        ),
    )
    return fn(H_v, bidx, zacc, zcnt, ones)


def _finalize_body(ps_ref, pc_ref, out_ref):
    sums = ps_ref[0] + ps_ref[1]
    counts = jnp.maximum((pc_ref[0] + pc_ref[1])[:, 0:1], 1.0)
    out_ref[...] = sums / counts


def kernel(H_v, batch):
    bidx = batch.reshape(NW, G, K)
    zacc = jnp.zeros((S, D), jnp.float32)
    zcnt = jnp.zeros((S, CW), jnp.float32)
    ones = jnp.ones((K, CW), jnp.float32)
    psum, pcnt = _sc_segment_sum(H_v, bidx, zacc, zcnt, ones)
    return pl.pallas_call(
        _finalize_body,
        out_shape=jax.ShapeDtypeStruct((S, D), jnp.float32),
    )(psum, pcnt)
